# baseline (speedup 1.0000x reference)
import functools

import jax
import jax.numpy as jnp
from jax import lax
from jax.experimental import pallas as pl
from jax.experimental.pallas import tpu as pltpu
from jax.experimental.pallas import tpu_sc as plsc

H = 512
W = 512
HD = 128
D = 2 * HD

TC_ROWS = 256
SC_ROWS = H - TC_ROWS
BH = 16

NC = 2
NS = 16
NW = NC * NS
RPW = SC_ROWS // NW
BR = 128
NCH = W // BR
NVEC = HD // 16

_mesh = plsc.VectorSubcoreMesh(core_axis_name="c", subcore_axis_name="s")


@functools.partial(
    pl.kernel,
    mesh=_mesh,
    out_type=jax.ShapeDtypeStruct((SC_ROWS, W, D), jnp.float32),
    scratch_types=[
        pltpu.VMEM((RPW, HD), jnp.float32),
        pltpu.VMEM((W, HD), jnp.float32),
        pltpu.VMEM((BR, HD), jnp.float32),
        pltpu.VMEM((BR, HD), jnp.float32),
        pltpu.SemaphoreType.DMA,
        pltpu.SemaphoreType.DMA,
        pltpu.SemaphoreType.DMA,
    ],
)
def _pe_sc(row_hbm, col_hbm, out_hbm, rows_v, col_v, blk_a, blk_b, sem_a,
           sem_b, sem_c):
    wid = lax.axis_index("s") * NC + lax.axis_index("c")
    base = wid * RPW
    pltpu.sync_copy(row_hbm.at[pl.ds(TC_ROWS + base, RPW)], rows_v)
    pltpu.sync_copy(col_hbm, col_v)

    blks = (blk_a, blk_b)
    sems = (sem_a, sem_b)
    pending = [None, None]
    col_pending = []
    for ii in range(RPW):
        b = ii % 2
        if pending[b] is not None:
            for hnd in pending[b]:
                hnd.wait()
        blk = blks[b]
        rv = [rows_v[ii, pl.ds(v * 16, 16)] for v in range(NVEC)]

        def fill(j, _, blk=blk, rv=rv):
            for v in range(NVEC):
                blk[j, pl.ds(v * 16, 16)] = rv[v]
            return 0

        lax.fori_loop(0, BR, fill, 0)
        r = base + ii
        col_pending.append(pltpu.async_copy(
            col_v, out_hbm.at[r, :, pl.ds(HD, HD)], sem_c))
        hs = []
        for c in range(NCH):
            hs.append(pltpu.async_copy(
                blk, out_hbm.at[r, pl.ds(c * BR, BR), pl.ds(0, HD)], sems[b]))
        pending[b] = hs
    for b in range(2):
        for hnd in pending[b]:
            hnd.wait()
    for hnd in col_pending:
        hnd.wait()


def _pe_tc(row_ref, col_ref, out_ref):
    r = row_ref[...]
    c = col_ref[...]
    out_ref[:, :, :HD] = jnp.broadcast_to(r[:, None, :], (BH, W, HD))
    out_ref[:, :, HD:] = jnp.broadcast_to(c[None, :, :], (BH, W, HD))


def kernel(row_embed, col_embed):
    top = pl.pallas_call(
        _pe_tc,
        out_shape=jax.ShapeDtypeStruct((TC_ROWS, W, D), jnp.float32),
        grid=(TC_ROWS // BH,),
        in_specs=[
            pl.BlockSpec((BH, HD), lambda i: (i, 0)),
            pl.BlockSpec((W, HD), lambda i: (0, 0)),
        ],
        out_specs=pl.BlockSpec((BH, W, D), lambda i: (i, 0, 0)),
    )(row_embed, col_embed)
    bot = _pe_sc(row_embed, col_embed)
    return jnp.concatenate([top, bot], axis=0).reshape(H * W, D)


# speedup vs baseline: 2.3881x; 2.3881x over previous
import functools

import jax
import jax.numpy as jnp
from jax import lax
from jax.experimental import pallas as pl
from jax.experimental.pallas import tpu as pltpu
from jax.experimental.pallas import tpu_sc as plsc
from jax._src.pallas import core as pallas_core
from jax._src.pallas import mpmd
from jax._src.pallas.mosaic import core as tpu_core

H = 512
W = 512
HD = 128
D = 2 * HD
NC = 2
NS = 16
NW = NC * NS
RPW = H // NW
BR = 128
NCH = W // BR
NVEC = HD // 16
SCS_ROWS = 12
SCS_DEPTH = 4

_scs_mesh = plsc.ScalarSubcoreMesh(axis_name="c", num_cores=NC)
_tec_mesh = plsc.VectorSubcoreMesh(core_axis_name="c", subcore_axis_name="s")


def _tec_vmem(shape):
    return pallas_core.CoreMemorySpace(
        tpu_core.MemorySpace.VMEM, _tec_mesh)(shape, jnp.float32)


def _scs_fn(row_hbm, col_hbm, out_hbm, rows_v, col_v, blk_a, blk_b, col_sp,
            sem_a, sem_b, sem_c, sem_s):
    del row_hbm, rows_v, col_v, blk_a, blk_b, sem_a, sem_b, sem_c
    c = lax.axis_index("c")
    pltpu.sync_copy(col_hbm, col_sp)
    hs = []
    for s in range(NS):
        base = (s * NC + c) * RPW
        for i in range(SCS_ROWS):
            if len(hs) >= SCS_DEPTH:
                hs.pop(0).wait()
            hs.append(pltpu.async_copy(
                col_sp, out_hbm.at[base + i, :, pl.ds(HD, HD)], sem_s))
    for hnd in hs:
        hnd.wait()


def _tec_fn(row_hbm, col_hbm, out_hbm, rows_v, col_v, blk_a, blk_b, col_sp,
            sem_a, sem_b, sem_c, sem_s):
    del col_sp, sem_s
    wid = lax.axis_index("s") * NC + lax.axis_index("c")
    base = wid * RPW
    pltpu.sync_copy(row_hbm.at[pl.ds(base, RPW)], rows_v)
    pltpu.sync_copy(col_hbm, col_v)

    blks = (blk_a, blk_b)
    sems = (sem_a, sem_b)
    pending = [None, None]
    col_pending = []
    for ii in range(RPW):
        b = ii % 2
        if pending[b] is not None:
            for hnd in pending[b]:
                hnd.wait()
        blk = blks[b]
        rv = [rows_v[ii, pl.ds(v * 16, 16)] for v in range(NVEC)]

        def fill(j, _, blk=blk, rv=rv):
            for v in range(NVEC):
                blk[j, pl.ds(v * 16, 16)] = rv[v]
            return 0

        lax.fori_loop(0, BR, fill, 0)
        r = base + ii
        if ii >= SCS_ROWS:
            col_pending.append(pltpu.async_copy(
                col_v, out_hbm.at[r, :, pl.ds(HD, HD)], sem_c))
        hs = []
        for ch in range(NCH):
            hs.append(pltpu.async_copy(
                blk, out_hbm.at[r, pl.ds(ch * BR, BR), pl.ds(0, HD)],
                sems[b]))
        pending[b] = hs
    for b in range(2):
        for hnd in pending[b]:
            hnd.wait()
    for hnd in col_pending:
        hnd.wait()


_pe_sc = mpmd.mpmd_map(
    [(_scs_mesh, _scs_fn), (_tec_mesh, _tec_fn)],
    out_types=[jax.ShapeDtypeStruct((H, W, D), jnp.float32)],
    scratch_types=[
        _tec_vmem((RPW, HD)),
        _tec_vmem((W, HD)),
        _tec_vmem((BR, HD)),
        _tec_vmem((BR, HD)),
        pltpu.VMEM_SHARED((W, HD), jnp.float32),
        pltpu.SemaphoreType.DMA @ _tec_mesh,
        pltpu.SemaphoreType.DMA @ _tec_mesh,
        pltpu.SemaphoreType.DMA @ _tec_mesh,
        pltpu.SemaphoreType.DMA @ _scs_mesh,
    ],
)


def kernel(row_embed, col_embed):
    [out] = _pe_sc(row_embed, col_embed)
    return out.reshape(H * W, D)


# speedup vs baseline: 2.4682x; 1.0336x over previous
import functools

import jax
import jax.numpy as jnp
from jax import lax
from jax.experimental import pallas as pl
from jax.experimental.pallas import tpu as pltpu
from jax.experimental.pallas import tpu_sc as plsc

H = 512
W = 512
HD = 128
D = 2 * HD
NC = 2
NS = 16
NW = NC * NS
RPW = H // NW
BR = 128
NCH = W // BR
NVEC = HD // 16

_mesh = plsc.VectorSubcoreMesh(core_axis_name="c", subcore_axis_name="s")


@functools.partial(
    pl.kernel,
    mesh=_mesh,
    out_type=jax.ShapeDtypeStruct((H, W, D), jnp.float32),
    scratch_types=[
        pltpu.VMEM((RPW, HD), jnp.float32),
        pltpu.VMEM((W, HD), jnp.float32),
        pltpu.VMEM((BR, HD), jnp.float32),
        pltpu.VMEM((BR, HD), jnp.float32),
        pltpu.SemaphoreType.DMA,
        pltpu.SemaphoreType.DMA,
        pltpu.SemaphoreType.DMA,
    ],
)
def _pe_sc(row_hbm, col_hbm, out_hbm, rows_v, col_v, blk_a, blk_b, sem_a,
           sem_b, sem_c):
    wid = lax.axis_index("s") * NC + lax.axis_index("c")
    base = wid * RPW
    pltpu.sync_copy(row_hbm.at[pl.ds(base, RPW)], rows_v)
    pltpu.sync_copy(col_hbm, col_v)

    blks = (blk_a, blk_b)
    sems = (sem_a, sem_b)
    pending = [None, None]
    col_pending = []
    for ii in range(RPW):
        b = ii % 2
        if pending[b] is not None:
            for hnd in pending[b]:
                hnd.wait()
        blk = blks[b]
        rv = [rows_v[ii, pl.ds(v * 16, 16)] for v in range(NVEC)]

        def fill(j, _, blk=blk, rv=rv):
            for v in range(NVEC):
                blk[j, pl.ds(v * 16, 16)] = rv[v]
            return 0

        lax.fori_loop(0, BR, fill, 0)
        r = base + ii
        col_pending.append(pltpu.async_copy(
            col_v, out_hbm.at[r, :, pl.ds(HD, HD)], sem_c))
        hs = []
        for c in range(NCH):
            hs.append(pltpu.async_copy(
                blk, out_hbm.at[r, pl.ds(c * BR, BR), pl.ds(0, HD)], sems[b]))
        pending[b] = hs
    for b in range(2):
        for hnd in pending[b]:
            hnd.wait()
    for hnd in col_pending:
        hnd.wait()


def kernel(row_embed, col_embed):
    return _pe_sc(row_embed, col_embed).reshape(H * W, D)
